# baseline (speedup 1.0000x reference)
import functools

import jax
import jax.numpy as jnp
from jax import lax
from jax.experimental import pallas as pl
from jax.experimental.pallas import tpu as pltpu
from jax.experimental.pallas import tpu_sc as plsc

BATCH = 16384
USER_DIM = 64
SMALL_DIM = 16
FC_IN = 128
OUT_DIM = 128

NC = 2
NS = 16
NW = NC * NS
B_PER_W = BATCH // NW


def _sc_gather(user_id, age, gender, occupation, zip_area,
               emb_users, bias_user, emb_age, emb_gender,
               emb_occupation, emb_zip_area):
  mesh = plsc.VectorSubcoreMesh(core_axis_name="c", subcore_axis_name="s",
                                num_cores=NC, num_subcores=NS)

  @functools.partial(
      pl.kernel,
      out_type=(
          jax.ShapeDtypeStruct((BATCH, USER_DIM), jnp.float32),
          jax.ShapeDtypeStruct((BATCH, SMALL_DIM), jnp.float32),
          jax.ShapeDtypeStruct((BATCH, SMALL_DIM), jnp.float32),
          jax.ShapeDtypeStruct((BATCH, SMALL_DIM), jnp.float32),
          jax.ShapeDtypeStruct((BATCH, SMALL_DIM), jnp.float32),
          jax.ShapeDtypeStruct((BATCH,), jnp.float32),
      ),
      mesh=mesh,
      compiler_params=pltpu.CompilerParams(use_tc_tiling_on_sc=False),
      scratch_types=[
          pltpu.VMEM((B_PER_W,), jnp.int32),
          pltpu.VMEM((B_PER_W,), jnp.int32),
          pltpu.VMEM((B_PER_W,), jnp.int32),
          pltpu.VMEM((B_PER_W,), jnp.int32),
          pltpu.VMEM((B_PER_W,), jnp.int32),
          pltpu.VMEM((B_PER_W, USER_DIM), jnp.float32),
          pltpu.VMEM((B_PER_W, SMALL_DIM), jnp.float32),
          pltpu.VMEM((B_PER_W, SMALL_DIM), jnp.float32),
          pltpu.VMEM((B_PER_W, SMALL_DIM), jnp.float32),
          pltpu.VMEM((B_PER_W, SMALL_DIM), jnp.float32),
          pltpu.VMEM((B_PER_W,), jnp.float32),
          pltpu.SemaphoreType.DMA,
      ],
  )
  def k(uid_h, age_h, gen_h, occ_h, zip_h,
        eu_h, bu_h, ea_h, eg_h, eo_h, ez_h,
        u_out, a_out, g_out, o_out, z_out, b_out,
        uidx, aidx, gidx, oidx, zidx,
        u_v, a_v, g_v, o_v, z_v, b_v, sem):
    wid = lax.axis_index("s") * NC + lax.axis_index("c")
    base = wid * B_PER_W
    pltpu.sync_copy(uid_h.at[pl.ds(base, B_PER_W)], uidx)
    pltpu.sync_copy(age_h.at[pl.ds(base, B_PER_W)], aidx)
    pltpu.sync_copy(gen_h.at[pl.ds(base, B_PER_W)], gidx)
    pltpu.sync_copy(occ_h.at[pl.ds(base, B_PER_W)], oidx)
    pltpu.sync_copy(zip_h.at[pl.ds(base, B_PER_W)], zidx)
    d0 = pltpu.async_copy(eu_h.at[uidx], u_v, sem)
    d1 = pltpu.async_copy(bu_h.at[uidx], b_v, sem)
    d2 = pltpu.async_copy(ea_h.at[aidx], a_v, sem)
    d3 = pltpu.async_copy(eg_h.at[gidx], g_v, sem)
    d4 = pltpu.async_copy(eo_h.at[oidx], o_v, sem)
    d5 = pltpu.async_copy(ez_h.at[zidx], z_v, sem)
    d0.wait(); d1.wait(); d2.wait(); d3.wait(); d4.wait(); d5.wait()
    pltpu.sync_copy(u_v, u_out.at[pl.ds(base, B_PER_W)])
    pltpu.sync_copy(a_v, a_out.at[pl.ds(base, B_PER_W)])
    pltpu.sync_copy(g_v, g_out.at[pl.ds(base, B_PER_W)])
    pltpu.sync_copy(o_v, o_out.at[pl.ds(base, B_PER_W)])
    pltpu.sync_copy(z_v, z_out.at[pl.ds(base, B_PER_W)])
    pltpu.sync_copy(b_v, b_out.at[pl.ds(base, B_PER_W)])

  return k(user_id, age, gender, occupation, zip_area,
           emb_users, bias_user.reshape(-1), emb_age, emb_gender,
           emb_occupation, emb_zip_area)


_BB = 2048


def _tc_head(u, a, g, o, z, fc_W, fc_b2):
  def body(u_ref, a_ref, g_ref, o_ref, z_ref, w_ref, b_ref, out_ref):
    x = jnp.concatenate(
        [u_ref[...], a_ref[...], g_ref[...], o_ref[...], z_ref[...]], axis=1)
    x = jnp.maximum(x, 0.0)
    y = lax.dot_general(x, w_ref[...], (((1,), (1,)), ((), ())),
                        preferred_element_type=jnp.float32)
    out_ref[...] = y + b_ref[...]

  return pl.pallas_call(
      body,
      grid=(BATCH // _BB,),
      in_specs=[
          pl.BlockSpec((_BB, USER_DIM), lambda i: (i, 0)),
          pl.BlockSpec((_BB, SMALL_DIM), lambda i: (i, 0)),
          pl.BlockSpec((_BB, SMALL_DIM), lambda i: (i, 0)),
          pl.BlockSpec((_BB, SMALL_DIM), lambda i: (i, 0)),
          pl.BlockSpec((_BB, SMALL_DIM), lambda i: (i, 0)),
          pl.BlockSpec((OUT_DIM, FC_IN), lambda i: (0, 0)),
          pl.BlockSpec((1, OUT_DIM), lambda i: (0, 0)),
      ],
      out_specs=pl.BlockSpec((_BB, OUT_DIM), lambda i: (i, 0)),
      out_shape=jax.ShapeDtypeStruct((BATCH, OUT_DIM), jnp.float32),
  )(u, a, g, o, z, fc_W, fc_b2)


def kernel(user_id, age, gender, occupation, zip_area, emb_users, bias_user,
           emb_age, emb_gender, emb_occupation, emb_zip_area, fc_W, fc_b):
  u, a, g, o, z, b = _sc_gather(user_id, age, gender, occupation, zip_area,
                                emb_users, bias_user, emb_age, emb_gender,
                                emb_occupation, emb_zip_area)
  user_vec = _tc_head(u, a, g, o, z, fc_W, fc_b.reshape(1, OUT_DIM))
  return (user_vec, b)


# speedup vs baseline: 1.8860x; 1.8860x over previous
import functools

import jax
import jax.numpy as jnp
from jax import lax
from jax.experimental import pallas as pl
from jax.experimental.pallas import tpu as pltpu
from jax.experimental.pallas import tpu_sc as plsc

BATCH = 16384
USER_DIM = 64
SMALL_DIM = 16
FC_IN = 128
OUT_DIM = 128
NUSERS = 1000000
HALF = NUSERS // 2

NC = 2
NS = 16
NW = NC * NS
B_PER_W = BATCH // NW
N_GRP = B_PER_W // 16


def _sc_bias(user_id, bias_flat):
  mesh = plsc.VectorSubcoreMesh(core_axis_name="c", subcore_axis_name="s",
                                num_cores=NC, num_subcores=NS)

  @functools.partial(
      pl.kernel,
      out_type=jax.ShapeDtypeStruct((BATCH,), jnp.float32),
      mesh=mesh,
      compiler_params=pltpu.CompilerParams(use_tc_tiling_on_sc=False),
      scratch_types=[
          pltpu.VMEM((B_PER_W,), jnp.int32),
          pltpu.VMEM((B_PER_W,), jnp.float32),
          pltpu.SemaphoreType.DMA,
      ],
  )
  def k(uid_h, bias_h, b_out, uidx, b_v, sem):
    wid = lax.axis_index("s") * NC + lax.axis_index("c")
    base = wid * B_PER_W
    pltpu.sync_copy(uid_h.at[pl.ds(base, B_PER_W)], uidx)
    pltpu.async_copy(bias_h.at[uidx], b_v, sem).wait()
    pltpu.sync_copy(b_v, b_out.at[pl.ds(base, B_PER_W)])

  return k(user_id, bias_flat)


_TC = 2048
_TWO_C = 2 * _TC
_NBLK = -(-NUSERS // _TWO_C)
N2 = _NBLK * _TC


def _tc_repack(ut):
  def body(x_ref, out_ref):
    x = x_ref[...]
    xa = jnp.transpose(x[:, :_TC], (1, 0))
    xb = jnp.transpose(x[:, _TC:], (1, 0))
    out_ref[...] = jnp.concatenate([xa, xb], axis=1)

  return pl.pallas_call(
      body,
      grid=(_NBLK,),
      in_specs=[pl.BlockSpec((USER_DIM, _TWO_C), lambda i: (0, i))],
      out_specs=pl.BlockSpec((_TC, FC_IN), lambda i: (i, 0)),
      out_shape=jax.ShapeDtypeStruct((N2, FC_IN), jnp.float32),
  )(ut)


def _sc_gather(idx2, age, gender, occupation, zip_area,
               u2, eat, egt, eot, ezt):
  mesh = plsc.VectorSubcoreMesh(core_axis_name="c", subcore_axis_name="s",
                                num_cores=NC, num_subcores=NS)

  @functools.partial(
      pl.kernel,
      out_type=(
          jax.ShapeDtypeStruct((BATCH, FC_IN), jnp.float32),
          jax.ShapeDtypeStruct((SMALL_DIM, BATCH), jnp.float32),
          jax.ShapeDtypeStruct((SMALL_DIM, BATCH), jnp.float32),
          jax.ShapeDtypeStruct((SMALL_DIM, BATCH), jnp.float32),
          jax.ShapeDtypeStruct((SMALL_DIM, BATCH), jnp.float32),
      ),
      mesh=mesh,
      compiler_params=pltpu.CompilerParams(needs_layout_passes=False),
      scratch_types=[
          pltpu.VMEM((B_PER_W,), jnp.int32),
          pltpu.VMEM((B_PER_W,), jnp.int32),
          pltpu.VMEM((B_PER_W,), jnp.int32),
          pltpu.VMEM((B_PER_W,), jnp.int32),
          pltpu.VMEM((B_PER_W,), jnp.int32),
          pltpu.VMEM((B_PER_W, FC_IN), jnp.float32),
          pltpu.VMEM((SMALL_DIM, B_PER_W), jnp.float32),
          pltpu.VMEM((SMALL_DIM, B_PER_W), jnp.float32),
          pltpu.VMEM((SMALL_DIM, B_PER_W), jnp.float32),
          pltpu.VMEM((SMALL_DIM, B_PER_W), jnp.float32),
          pltpu.VMEM((SMALL_DIM, 8), jnp.float32),
          pltpu.VMEM((SMALL_DIM, 3), jnp.float32),
          pltpu.VMEM((SMALL_DIM, 64), jnp.float32),
          pltpu.VMEM((SMALL_DIM, 1000), jnp.float32),
          pltpu.SemaphoreType.DMA,
      ],
  )
  def k(idx_h, age_h, gen_h, occ_h, zip_h,
        u2_h, eat_h, egt_h, eot_h, ezt_h,
        p_out, a_out, g_out, o_out, z_out,
        uidx, aidx, gidx, oidx, zidx,
        p_v, a_v, g_v, o_v, z_v,
        ea_v, eg_v, eo_v, ez_v, sem):
    wid = lax.axis_index("s") * NC + lax.axis_index("c")
    base = wid * B_PER_W
    pltpu.sync_copy(idx_h.at[pl.ds(base, B_PER_W)], uidx)
    pltpu.sync_copy(age_h.at[pl.ds(base, B_PER_W)], aidx)
    pltpu.sync_copy(gen_h.at[pl.ds(base, B_PER_W)], gidx)
    pltpu.sync_copy(occ_h.at[pl.ds(base, B_PER_W)], oidx)
    pltpu.sync_copy(zip_h.at[pl.ds(base, B_PER_W)], zidx)
    pltpu.sync_copy(eat_h, ea_v)
    pltpu.sync_copy(egt_h, eg_v)
    pltpu.sync_copy(eot_h, eo_v)
    pltpu.sync_copy(ezt_h, ez_v)

    du = pltpu.async_copy(u2_h.at[uidx], p_v, sem)

    def grp(g, carry):
      s = g * 16
      ia = aidx[pl.ds(s, 16)]
      ig = gidx[pl.ds(s, 16)]
      io = oidx[pl.ds(s, 16)]
      iz = zidx[pl.ds(s, 16)]
      for f in range(SMALL_DIM):
        fv = jnp.full((16,), f, jnp.int32)
        a_v[f, pl.ds(s, 16)] = plsc.load_gather(ea_v, [fv, ia])
        g_v[f, pl.ds(s, 16)] = plsc.load_gather(eg_v, [fv, ig])
        o_v[f, pl.ds(s, 16)] = plsc.load_gather(eo_v, [fv, io])
        z_v[f, pl.ds(s, 16)] = plsc.load_gather(ez_v, [fv, iz])
      return carry

    lax.fori_loop(0, N_GRP, grp, 0, unroll=False)
    du.wait()

    pltpu.sync_copy(p_v, p_out.at[pl.ds(base, B_PER_W)])
    pltpu.sync_copy(a_v, a_out.at[:, pl.ds(base, B_PER_W)])
    pltpu.sync_copy(g_v, g_out.at[:, pl.ds(base, B_PER_W)])
    pltpu.sync_copy(o_v, o_out.at[:, pl.ds(base, B_PER_W)])
    pltpu.sync_copy(z_v, z_out.at[:, pl.ds(base, B_PER_W)])

  return k(idx2, age, gender, occupation, zip_area, u2, eat, egt, eot, ezt)


_BB = 2048


def _tc_head(pairs, parityf, at_g, gt_g, ot_g, zt_g, fc_W, fc_b2):
  def body(p_ref, pf_ref, a_ref, g_ref, o_ref, z_ref,
           w_ref, b_ref, out_ref):
    pr = jnp.maximum(p_ref[...], 0.0)
    pf = pf_ref[...]
    u = pr[:, :USER_DIM] + pf * (pr[:, USER_DIM:] - pr[:, :USER_DIM])
    w = w_ref[...]
    yu = lax.dot_general(u, w[:, :USER_DIM], (((1,), (1,)), ((), ())),
                         preferred_element_type=jnp.float32)
    st = jnp.concatenate(
        [a_ref[...], g_ref[...], o_ref[...], z_ref[...]], axis=0)
    st = jnp.maximum(st, 0.0)
    ys = lax.dot_general(st, w[:, USER_DIM:], (((0,), (1,)), ((), ())),
                         preferred_element_type=jnp.float32)
    out_ref[...] = yu + ys + b_ref[...]

  return pl.pallas_call(
      body,
      grid=(BATCH // _BB,),
      in_specs=[
          pl.BlockSpec((_BB, FC_IN), lambda i: (i, 0)),
          pl.BlockSpec((_BB, 1), lambda i: (i, 0)),
          pl.BlockSpec((SMALL_DIM, _BB), lambda i: (0, i)),
          pl.BlockSpec((SMALL_DIM, _BB), lambda i: (0, i)),
          pl.BlockSpec((SMALL_DIM, _BB), lambda i: (0, i)),
          pl.BlockSpec((SMALL_DIM, _BB), lambda i: (0, i)),
          pl.BlockSpec((OUT_DIM, FC_IN), lambda i: (0, 0)),
          pl.BlockSpec((1, OUT_DIM), lambda i: (0, 0)),
      ],
      out_specs=pl.BlockSpec((_BB, OUT_DIM), lambda i: (i, 0)),
      out_shape=jax.ShapeDtypeStruct((BATCH, OUT_DIM), jnp.float32),
  )(pairs, parityf, at_g, gt_g, ot_g, zt_g, fc_W, fc_b2)


def kernel(user_id, age, gender, occupation, zip_area, emb_users, bias_user,
           emb_age, emb_gender, emb_occupation, emb_zip_area, fc_W, fc_b):
  b = _sc_bias(user_id, bias_user.reshape(-1))
  u2 = _tc_repack(emb_users.T)
  rr = user_id % _TWO_C
  idx2 = (user_id // _TWO_C) * _TC + (rr % _TC)
  parityf = (rr >= _TC).astype(jnp.float32).reshape(BATCH, 1)
  pairs, at_g, gt_g, ot_g, zt_g = _sc_gather(
      idx2, age, gender, occupation, zip_area, u2,
      emb_age.T, emb_gender.T, emb_occupation.T, emb_zip_area.T)
  user_vec = _tc_head(pairs, parityf, at_g, gt_g, ot_g, zt_g,
                      fc_W, fc_b.reshape(1, OUT_DIM))
  return (user_vec, b)


# speedup vs baseline: 1.8954x; 1.0050x over previous
import functools

import jax
import jax.numpy as jnp
from jax import lax
from jax.experimental import pallas as pl
from jax.experimental.pallas import tpu as pltpu
from jax.experimental.pallas import tpu_sc as plsc

BATCH = 16384
USER_DIM = 64
SMALL_DIM = 16
FC_IN = 128
OUT_DIM = 128
NUSERS = 1000000
HALF = NUSERS // 2

NC = 2
NS = 16
NW = NC * NS
B_PER_W = BATCH // NW
N_GRP = B_PER_W // 16


def _sc_bias(user_id, bias_flat):
  mesh = plsc.VectorSubcoreMesh(core_axis_name="c", subcore_axis_name="s",
                                num_cores=NC, num_subcores=NS)

  @functools.partial(
      pl.kernel,
      out_type=jax.ShapeDtypeStruct((BATCH,), jnp.float32),
      mesh=mesh,
      compiler_params=pltpu.CompilerParams(use_tc_tiling_on_sc=False),
      scratch_types=[
          pltpu.VMEM((B_PER_W,), jnp.int32),
          pltpu.VMEM((B_PER_W,), jnp.float32),
          pltpu.SemaphoreType.DMA,
      ],
  )
  def k(uid_h, bias_h, b_out, uidx, b_v, sem):
    wid = lax.axis_index("s") * NC + lax.axis_index("c")
    base = wid * B_PER_W
    pltpu.sync_copy(uid_h.at[pl.ds(base, B_PER_W)], uidx)
    pltpu.async_copy(bias_h.at[uidx], b_v, sem).wait()
    pltpu.sync_copy(b_v, b_out.at[pl.ds(base, B_PER_W)])

  return k(user_id, bias_flat)


_QC = 2048
_BCOLS = 4 * _QC
_NBLK = -(-NUSERS // _BCOLS)
N4 = _NBLK * _QC
_HD = USER_DIM // 2


def _tc_repack(ut):
  def body(x_ref, out_ref):
    x = x_ref[...]
    parts = []
    for q in range(4):
      t = jnp.transpose(x[:, q * _QC:(q + 1) * _QC], (1, 0))
      lo = lax.bitcast_convert_type(
          t[:, :_HD].astype(jnp.bfloat16), jnp.uint16).astype(jnp.uint32)
      hi = lax.bitcast_convert_type(
          t[:, _HD:].astype(jnp.bfloat16), jnp.uint16).astype(jnp.uint32)
      parts.append(lax.bitcast_convert_type((hi << 16) | lo, jnp.float32))
    out_ref[...] = jnp.concatenate(parts, axis=1)

  return pl.pallas_call(
      body,
      grid=(_NBLK,),
      in_specs=[pl.BlockSpec((USER_DIM, _BCOLS), lambda i: (0, i))],
      out_specs=pl.BlockSpec((_QC, FC_IN), lambda i: (i, 0)),
      out_shape=jax.ShapeDtypeStruct((N4, FC_IN), jnp.float32),
  )(ut)


def _sc_gather(idx2, age, gender, occupation, zip_area,
               u2, eat, egt, eot, ezt):
  mesh = plsc.VectorSubcoreMesh(core_axis_name="c", subcore_axis_name="s",
                                num_cores=NC, num_subcores=NS)

  @functools.partial(
      pl.kernel,
      out_type=(
          jax.ShapeDtypeStruct((BATCH, FC_IN), jnp.float32),
          jax.ShapeDtypeStruct((SMALL_DIM, BATCH), jnp.float32),
          jax.ShapeDtypeStruct((SMALL_DIM, BATCH), jnp.float32),
          jax.ShapeDtypeStruct((SMALL_DIM, BATCH), jnp.float32),
          jax.ShapeDtypeStruct((SMALL_DIM, BATCH), jnp.float32),
      ),
      mesh=mesh,
      compiler_params=pltpu.CompilerParams(needs_layout_passes=False),
      scratch_types=[
          pltpu.VMEM((B_PER_W,), jnp.int32),
          pltpu.VMEM((B_PER_W,), jnp.int32),
          pltpu.VMEM((B_PER_W,), jnp.int32),
          pltpu.VMEM((B_PER_W,), jnp.int32),
          pltpu.VMEM((B_PER_W,), jnp.int32),
          pltpu.VMEM((B_PER_W, FC_IN), jnp.float32),
          pltpu.VMEM((SMALL_DIM, B_PER_W), jnp.float32),
          pltpu.VMEM((SMALL_DIM, B_PER_W), jnp.float32),
          pltpu.VMEM((SMALL_DIM, B_PER_W), jnp.float32),
          pltpu.VMEM((SMALL_DIM, B_PER_W), jnp.float32),
          pltpu.VMEM((SMALL_DIM, 8), jnp.float32),
          pltpu.VMEM((SMALL_DIM, 3), jnp.float32),
          pltpu.VMEM((SMALL_DIM, 64), jnp.float32),
          pltpu.VMEM((SMALL_DIM, 1000), jnp.float32),
          pltpu.SemaphoreType.DMA,
      ],
  )
  def k(idx_h, age_h, gen_h, occ_h, zip_h,
        u2_h, eat_h, egt_h, eot_h, ezt_h,
        p_out, a_out, g_out, o_out, z_out,
        uidx, aidx, gidx, oidx, zidx,
        p_v, a_v, g_v, o_v, z_v,
        ea_v, eg_v, eo_v, ez_v, sem):
    wid = lax.axis_index("s") * NC + lax.axis_index("c")
    base = wid * B_PER_W
    pltpu.sync_copy(idx_h.at[pl.ds(base, B_PER_W)], uidx)
    pltpu.sync_copy(age_h.at[pl.ds(base, B_PER_W)], aidx)
    pltpu.sync_copy(gen_h.at[pl.ds(base, B_PER_W)], gidx)
    pltpu.sync_copy(occ_h.at[pl.ds(base, B_PER_W)], oidx)
    pltpu.sync_copy(zip_h.at[pl.ds(base, B_PER_W)], zidx)
    pltpu.sync_copy(eat_h, ea_v)
    pltpu.sync_copy(egt_h, eg_v)
    pltpu.sync_copy(eot_h, eo_v)
    pltpu.sync_copy(ezt_h, ez_v)

    du = pltpu.async_copy(u2_h.at[uidx], p_v, sem)

    def grp(g, carry):
      s = g * 16
      ia = aidx[pl.ds(s, 16)]
      ig = gidx[pl.ds(s, 16)]
      io = oidx[pl.ds(s, 16)]
      iz = zidx[pl.ds(s, 16)]
      for f in range(SMALL_DIM):
        fv = jnp.full((16,), f, jnp.int32)
        a_v[f, pl.ds(s, 16)] = plsc.load_gather(ea_v, [fv, ia])
        g_v[f, pl.ds(s, 16)] = plsc.load_gather(eg_v, [fv, ig])
        o_v[f, pl.ds(s, 16)] = plsc.load_gather(eo_v, [fv, io])
        z_v[f, pl.ds(s, 16)] = plsc.load_gather(ez_v, [fv, iz])
      return carry

    lax.fori_loop(0, N_GRP, grp, 0, unroll=False)
    du.wait()

    pltpu.sync_copy(p_v, p_out.at[pl.ds(base, B_PER_W)])
    pltpu.sync_copy(a_v, a_out.at[:, pl.ds(base, B_PER_W)])
    pltpu.sync_copy(g_v, g_out.at[:, pl.ds(base, B_PER_W)])
    pltpu.sync_copy(o_v, o_out.at[:, pl.ds(base, B_PER_W)])
    pltpu.sync_copy(z_v, z_out.at[:, pl.ds(base, B_PER_W)])

  return k(idx2, age, gender, occupation, zip_area, u2, eat, egt, eot, ezt)


_BB = 2048


def _tc_head(pairs, quarterf, at_g, gt_g, ot_g, zt_g, fc_W, fc_b2):
  def body(p_ref, qf_ref, a_ref, g_ref, o_ref, z_ref,
           w_ref, b_ref, out_ref):
    pw = lax.bitcast_convert_type(p_ref[...], jnp.uint32)
    qf = qf_ref[...]
    sel = jnp.where(
        qf[:, 0:1] > 0.5, pw[:, :_HD],
        jnp.where(qf[:, 1:2] > 0.5, pw[:, _HD:2 * _HD],
                  jnp.where(qf[:, 2:3] > 0.5, pw[:, 2 * _HD:3 * _HD],
                            pw[:, 3 * _HD:])))
    lo = lax.bitcast_convert_type(
        (sel & 0xFFFF).astype(jnp.uint16), jnp.bfloat16).astype(jnp.float32)
    hi = lax.bitcast_convert_type(
        (sel >> 16).astype(jnp.uint16), jnp.bfloat16).astype(jnp.float32)
    u = jnp.maximum(jnp.concatenate([lo, hi], axis=1), 0.0)
    w = w_ref[...]
    yu = lax.dot_general(u, w[:, :USER_DIM], (((1,), (1,)), ((), ())),
                         preferred_element_type=jnp.float32)
    st = jnp.concatenate(
        [a_ref[...], g_ref[...], o_ref[...], z_ref[...]], axis=0)
    st = jnp.maximum(st, 0.0)
    ys = lax.dot_general(st, w[:, USER_DIM:], (((0,), (1,)), ((), ())),
                         preferred_element_type=jnp.float32)
    out_ref[...] = yu + ys + b_ref[...]

  return pl.pallas_call(
      body,
      grid=(BATCH // _BB,),
      in_specs=[
          pl.BlockSpec((_BB, FC_IN), lambda i: (i, 0)),
          pl.BlockSpec((_BB, 4), lambda i: (i, 0)),
          pl.BlockSpec((SMALL_DIM, _BB), lambda i: (0, i)),
          pl.BlockSpec((SMALL_DIM, _BB), lambda i: (0, i)),
          pl.BlockSpec((SMALL_DIM, _BB), lambda i: (0, i)),
          pl.BlockSpec((SMALL_DIM, _BB), lambda i: (0, i)),
          pl.BlockSpec((OUT_DIM, FC_IN), lambda i: (0, 0)),
          pl.BlockSpec((1, OUT_DIM), lambda i: (0, 0)),
      ],
      out_specs=pl.BlockSpec((_BB, OUT_DIM), lambda i: (i, 0)),
      out_shape=jax.ShapeDtypeStruct((BATCH, OUT_DIM), jnp.float32),
  )(pairs, quarterf, at_g, gt_g, ot_g, zt_g, fc_W, fc_b2)


def kernel(user_id, age, gender, occupation, zip_area, emb_users, bias_user,
           emb_age, emb_gender, emb_occupation, emb_zip_area, fc_W, fc_b):
  b = _sc_bias(user_id, bias_user.reshape(-1))
  u4 = _tc_repack(emb_users.T)
  rr = user_id % _BCOLS
  idx4 = (user_id // _BCOLS) * _QC + (rr % _QC)
  q = rr // _QC
  quarterf = (q[:, None] == jnp.arange(4)[None, :]).astype(jnp.float32)
  pairs, at_g, gt_g, ot_g, zt_g = _sc_gather(
      idx4, age, gender, occupation, zip_area, u4,
      emb_age.T, emb_gender.T, emb_occupation.T, emb_zip_area.T)
  user_vec = _tc_head(pairs, quarterf, at_g, gt_g, ot_g, zt_g,
                      fc_W, fc_b.reshape(1, OUT_DIM))
  return (user_vec, b)


# speedup vs baseline: 2.2195x; 1.1710x over previous
import functools

import jax
import jax.numpy as jnp
from jax import lax
from jax.experimental import pallas as pl
from jax.experimental.pallas import tpu as pltpu
from jax.experimental.pallas import tpu_sc as plsc

BATCH = 16384
USER_DIM = 64
SMALL_DIM = 16
FC_IN = 128
OUT_DIM = 128
NUSERS = 1000000
HALF = NUSERS // 2

NC = 2
NS = 16
NW = NC * NS
B_PER_W = BATCH // NW
N_GRP = B_PER_W // 16


def _sc_bias(user_id, bias_flat):
  mesh = plsc.VectorSubcoreMesh(core_axis_name="c", subcore_axis_name="s",
                                num_cores=NC, num_subcores=NS)

  @functools.partial(
      pl.kernel,
      out_type=jax.ShapeDtypeStruct((BATCH,), jnp.float32),
      mesh=mesh,
      compiler_params=pltpu.CompilerParams(use_tc_tiling_on_sc=False),
      scratch_types=[
          pltpu.VMEM((B_PER_W,), jnp.int32),
          pltpu.VMEM((B_PER_W,), jnp.float32),
          pltpu.SemaphoreType.DMA,
      ],
  )
  def k(uid_h, bias_h, b_out, uidx, b_v, sem):
    wid = lax.axis_index("s") * NC + lax.axis_index("c")
    base = wid * B_PER_W
    pltpu.sync_copy(uid_h.at[pl.ds(base, B_PER_W)], uidx)
    pltpu.async_copy(bias_h.at[uidx], b_v, sem).wait()
    pltpu.sync_copy(b_v, b_out.at[pl.ds(base, B_PER_W)])

  return k(user_id, bias_flat)


_QC = 2048
_BCOLS = 4 * _QC
_NBLK = -(-NUSERS // _BCOLS)
N4 = _NBLK * _QC
_HD = USER_DIM // 2


def _tc_repack(ut):
  def body(x_ref, out_ref):
    x = x_ref[...]
    lo = lax.bitcast_convert_type(
        x[:_HD, :].astype(jnp.bfloat16), jnp.uint16).astype(jnp.uint32)
    hi = lax.bitcast_convert_type(
        x[_HD:, :].astype(jnp.bfloat16), jnp.uint16).astype(jnp.uint32)
    packed = lax.bitcast_convert_type((hi << 16) | lo, jnp.float32)
    parts = [
        jnp.transpose(packed[:, q * _QC:(q + 1) * _QC], (1, 0))
        for q in range(4)
    ]
    out_ref[...] = jnp.concatenate(parts, axis=1)

  return pl.pallas_call(
      body,
      grid=(_NBLK,),
      in_specs=[pl.BlockSpec((USER_DIM, _BCOLS), lambda i: (0, i))],
      out_specs=pl.BlockSpec((_QC, FC_IN), lambda i: (i, 0)),
      out_shape=jax.ShapeDtypeStruct((N4, FC_IN), jnp.float32),
  )(ut)


def _sc_gather(idx2, age, gender, occupation, zip_area,
               u2, eat, egt, eot, ezt):
  mesh = plsc.VectorSubcoreMesh(core_axis_name="c", subcore_axis_name="s",
                                num_cores=NC, num_subcores=NS)

  @functools.partial(
      pl.kernel,
      out_type=(
          jax.ShapeDtypeStruct((BATCH, FC_IN), jnp.float32),
          jax.ShapeDtypeStruct((SMALL_DIM, BATCH), jnp.float32),
          jax.ShapeDtypeStruct((SMALL_DIM, BATCH), jnp.float32),
          jax.ShapeDtypeStruct((SMALL_DIM, BATCH), jnp.float32),
          jax.ShapeDtypeStruct((SMALL_DIM, BATCH), jnp.float32),
      ),
      mesh=mesh,
      compiler_params=pltpu.CompilerParams(needs_layout_passes=False),
      scratch_types=[
          pltpu.VMEM((B_PER_W,), jnp.int32),
          pltpu.VMEM((B_PER_W,), jnp.int32),
          pltpu.VMEM((B_PER_W,), jnp.int32),
          pltpu.VMEM((B_PER_W,), jnp.int32),
          pltpu.VMEM((B_PER_W,), jnp.int32),
          pltpu.VMEM((B_PER_W, FC_IN), jnp.float32),
          pltpu.VMEM((SMALL_DIM, B_PER_W), jnp.float32),
          pltpu.VMEM((SMALL_DIM, B_PER_W), jnp.float32),
          pltpu.VMEM((SMALL_DIM, B_PER_W), jnp.float32),
          pltpu.VMEM((SMALL_DIM, B_PER_W), jnp.float32),
          pltpu.VMEM((SMALL_DIM, 8), jnp.float32),
          pltpu.VMEM((SMALL_DIM, 3), jnp.float32),
          pltpu.VMEM((SMALL_DIM, 64), jnp.float32),
          pltpu.VMEM((SMALL_DIM, 1000), jnp.float32),
          pltpu.SemaphoreType.DMA,
      ],
  )
  def k(idx_h, age_h, gen_h, occ_h, zip_h,
        u2_h, eat_h, egt_h, eot_h, ezt_h,
        p_out, a_out, g_out, o_out, z_out,
        uidx, aidx, gidx, oidx, zidx,
        p_v, a_v, g_v, o_v, z_v,
        ea_v, eg_v, eo_v, ez_v, sem):
    wid = lax.axis_index("s") * NC + lax.axis_index("c")
    base = wid * B_PER_W
    pltpu.sync_copy(idx_h.at[pl.ds(base, B_PER_W)], uidx)
    pltpu.sync_copy(age_h.at[pl.ds(base, B_PER_W)], aidx)
    pltpu.sync_copy(gen_h.at[pl.ds(base, B_PER_W)], gidx)
    pltpu.sync_copy(occ_h.at[pl.ds(base, B_PER_W)], oidx)
    pltpu.sync_copy(zip_h.at[pl.ds(base, B_PER_W)], zidx)
    pltpu.sync_copy(eat_h, ea_v)
    pltpu.sync_copy(egt_h, eg_v)
    pltpu.sync_copy(eot_h, eo_v)
    pltpu.sync_copy(ezt_h, ez_v)

    du = pltpu.async_copy(u2_h.at[uidx], p_v, sem)

    def grp(g, carry):
      s = g * 16
      ia = aidx[pl.ds(s, 16)]
      ig = gidx[pl.ds(s, 16)]
      io = oidx[pl.ds(s, 16)]
      iz = zidx[pl.ds(s, 16)]
      for f in range(SMALL_DIM):
        fv = jnp.full((16,), f, jnp.int32)
        a_v[f, pl.ds(s, 16)] = plsc.load_gather(ea_v, [fv, ia])
        g_v[f, pl.ds(s, 16)] = plsc.load_gather(eg_v, [fv, ig])
        o_v[f, pl.ds(s, 16)] = plsc.load_gather(eo_v, [fv, io])
        z_v[f, pl.ds(s, 16)] = plsc.load_gather(ez_v, [fv, iz])
      return carry

    lax.fori_loop(0, N_GRP, grp, 0, unroll=False)
    du.wait()

    pltpu.sync_copy(p_v, p_out.at[pl.ds(base, B_PER_W)])
    pltpu.sync_copy(a_v, a_out.at[:, pl.ds(base, B_PER_W)])
    pltpu.sync_copy(g_v, g_out.at[:, pl.ds(base, B_PER_W)])
    pltpu.sync_copy(o_v, o_out.at[:, pl.ds(base, B_PER_W)])
    pltpu.sync_copy(z_v, z_out.at[:, pl.ds(base, B_PER_W)])

  return k(idx2, age, gender, occupation, zip_area, u2, eat, egt, eot, ezt)


_BB = 2048


def _tc_head(pairs, quarterf, at_g, gt_g, ot_g, zt_g, fc_W, fc_b2):
  def body(p_ref, qf_ref, a_ref, g_ref, o_ref, z_ref,
           w_ref, b_ref, out_ref):
    pw = lax.bitcast_convert_type(p_ref[...], jnp.uint32)
    qf = qf_ref[...]
    sel = jnp.where(
        qf[:, 0:1] > 0.5, pw[:, :_HD],
        jnp.where(qf[:, 1:2] > 0.5, pw[:, _HD:2 * _HD],
                  jnp.where(qf[:, 2:3] > 0.5, pw[:, 2 * _HD:3 * _HD],
                            pw[:, 3 * _HD:])))
    lo = lax.bitcast_convert_type(
        (sel & 0xFFFF).astype(jnp.uint16), jnp.bfloat16).astype(jnp.float32)
    hi = lax.bitcast_convert_type(
        (sel >> 16).astype(jnp.uint16), jnp.bfloat16).astype(jnp.float32)
    u = jnp.maximum(jnp.concatenate([lo, hi], axis=1), 0.0)
    w = w_ref[...]
    yu = lax.dot_general(u, w[:, :USER_DIM], (((1,), (1,)), ((), ())),
                         preferred_element_type=jnp.float32)
    st = jnp.concatenate(
        [a_ref[...], g_ref[...], o_ref[...], z_ref[...]], axis=0)
    st = jnp.maximum(st, 0.0)
    ys = lax.dot_general(st, w[:, USER_DIM:], (((0,), (1,)), ((), ())),
                         preferred_element_type=jnp.float32)
    out_ref[...] = yu + ys + b_ref[...]

  return pl.pallas_call(
      body,
      grid=(BATCH // _BB,),
      in_specs=[
          pl.BlockSpec((_BB, FC_IN), lambda i: (i, 0)),
          pl.BlockSpec((_BB, 4), lambda i: (i, 0)),
          pl.BlockSpec((SMALL_DIM, _BB), lambda i: (0, i)),
          pl.BlockSpec((SMALL_DIM, _BB), lambda i: (0, i)),
          pl.BlockSpec((SMALL_DIM, _BB), lambda i: (0, i)),
          pl.BlockSpec((SMALL_DIM, _BB), lambda i: (0, i)),
          pl.BlockSpec((OUT_DIM, FC_IN), lambda i: (0, 0)),
          pl.BlockSpec((1, OUT_DIM), lambda i: (0, 0)),
      ],
      out_specs=pl.BlockSpec((_BB, OUT_DIM), lambda i: (i, 0)),
      out_shape=jax.ShapeDtypeStruct((BATCH, OUT_DIM), jnp.float32),
  )(pairs, quarterf, at_g, gt_g, ot_g, zt_g, fc_W, fc_b2)


def kernel(user_id, age, gender, occupation, zip_area, emb_users, bias_user,
           emb_age, emb_gender, emb_occupation, emb_zip_area, fc_W, fc_b):
  b = _sc_bias(user_id, bias_user.reshape(-1))
  u4 = _tc_repack(emb_users.T)
  rr = user_id % _BCOLS
  idx4 = (user_id // _BCOLS) * _QC + (rr % _QC)
  q = rr // _QC
  quarterf = (q[:, None] == jnp.arange(4)[None, :]).astype(jnp.float32)
  pairs, at_g, gt_g, ot_g, zt_g = _sc_gather(
      idx4, age, gender, occupation, zip_area, u4,
      emb_age.T, emb_gender.T, emb_occupation.T, emb_zip_area.T)
  user_vec = _tc_head(pairs, quarterf, at_g, gt_g, ot_g, zt_g,
                      fc_W, fc_b.reshape(1, OUT_DIM))
  return (user_vec, b)
